# Initial kernel scaffold; baseline (speedup 1.0000x reference)
import jax
import jax.numpy as jnp
from jax.experimental import pallas as pl


def kernel(entity_tok, table):
    raise NotImplementedError("write your pallas kernel here")


# speedup vs baseline: 1.1070x; 1.1070x over previous
import functools

import jax
import jax.numpy as jnp
from jax import lax
from jax.experimental import pallas as pl
from jax.experimental.pallas import tpu as pltpu
from jax.experimental.pallas import tpu_sc as plsc

NC = 2
NS = 16
NW = NC * NS
G = 128


@functools.lru_cache(maxsize=None)
def _make_gather(vocab: int, embed: int, b_total: int):
    assert b_total % (NW * G) == 0
    b_per_w = b_total // NW
    rows_per_w = b_per_w // G
    k = 8
    assert rows_per_w % k == 0
    c = k * G
    n_chunks = rows_per_w // k

    mesh = plsc.VectorSubcoreMesh(core_axis_name="c", subcore_axis_name="s")

    @functools.partial(
        pl.kernel,
        mesh=mesh,
        compiler_params=pltpu.CompilerParams(use_tc_tiling_on_sc=False),
        out_type=jax.ShapeDtypeStruct((b_total, embed), jnp.float32),
        scratch_types=[
            pltpu.VMEM((2, k, G), jnp.int32),
            pltpu.VMEM((2, c, embed), jnp.float32),
            pltpu.SemaphoreType.DMA,
            pltpu.SemaphoreType.DMA,
            pltpu.SemaphoreType.DMA,
            pltpu.SemaphoreType.DMA,
        ],
    )
    def gather_kernel(idx_hbm, table_hbm, out_hbm, idx_v, rows_v,
                      gsem0, gsem1, osem0, osem1):
        gsem = (gsem0, gsem1)
        osem = (osem0, osem1)
        wid = lax.axis_index("s") * NC + lax.axis_index("c")
        row0 = wid * rows_per_w
        out0 = wid * b_per_w

        def load_chunk(ch, buf):
            pltpu.sync_copy(idx_hbm.at[pl.ds(row0 + ch * k, k)],
                            idx_v.at[buf])
            for j in range(k):
                pltpu.async_copy(table_hbm.at[idx_v.at[buf, j]],
                                 rows_v.at[buf, pl.ds(j * G, G)],
                                 gsem[buf])

        def drain_gathers(buf):
            for j in range(k):
                pltpu.make_async_copy(table_hbm.at[idx_v.at[buf, j]],
                                      rows_v.at[buf, pl.ds(j * G, G)],
                                      gsem[buf]).wait()

        def start_store(ch, buf):
            return pltpu.async_copy(rows_v.at[buf],
                                    out_hbm.at[pl.ds(out0 + ch * c, c)],
                                    osem[buf])

        def wait_store(ch, buf):
            pltpu.make_async_copy(rows_v.at[buf],
                                  out_hbm.at[pl.ds(out0 + ch * c, c)],
                                  osem[buf]).wait()

        def block(ch, b):
            wait_store(ch - 2, b)
            load_chunk(ch, b)
            drain_gathers(1 - b)
            start_store(ch - 1, 1 - b)

        load_chunk(0, 0)
        load_chunk(1, 1)
        drain_gathers(0)
        start_store(0, 0)

        n_even = n_chunks if n_chunks % 2 == 0 else n_chunks - 1

        def body(g):
            block(g, 0)
            block(g + 1, 1)

        pl.loop(2, n_even, step=2)(body)

        if n_chunks % 2:
            block(n_chunks - 1, 0)

        last = n_chunks - 1
        bl = last & 1
        wait_store(last - 1, 1 - bl)
        drain_gathers(bl)
        start_store(last, bl)
        wait_store(last, bl)

    return gather_kernel


def kernel(entity_tok, table):
    batch, hist = entity_tok.shape
    vocab, embed = table.shape
    b_total = batch * hist
    idx = entity_tok.reshape(b_total // G, G).astype(jnp.int32)
    out = _make_gather(vocab, embed, b_total)(idx, table)
    return out.reshape(batch, hist, embed)


# speedup vs baseline: 1.1070x; 1.0000x over previous
import functools

import jax
import jax.numpy as jnp
from jax import lax
from jax.experimental import pallas as pl
from jax.experimental.pallas import tpu as pltpu
from jax.experimental.pallas import tpu_sc as plsc

NC = 2
NS = 16
NW = NC * NS
G = 128


@functools.lru_cache(maxsize=None)
def _make_gather(vocab: int, embed: int, b_total: int):
    assert b_total % (NW * G) == 0
    b_per_w = b_total // NW
    rows_per_w = b_per_w // G
    k = 8
    assert rows_per_w % k == 0
    c = k * G
    n_chunks = rows_per_w // k

    mesh = plsc.VectorSubcoreMesh(core_axis_name="c", subcore_axis_name="s")

    @functools.partial(
        pl.kernel,
        mesh=mesh,
        compiler_params=pltpu.CompilerParams(use_tc_tiling_on_sc=False),
        out_type=jax.ShapeDtypeStruct((b_total, embed), jnp.float32),
        scratch_types=[
            pltpu.VMEM((2, c), jnp.int32),
            pltpu.VMEM((2, c, embed), jnp.float32),
            pltpu.SemaphoreType.DMA,
            pltpu.SemaphoreType.DMA,
            pltpu.SemaphoreType.DMA,
            pltpu.SemaphoreType.DMA,
        ],
    )
    def gather_kernel(idx_hbm, table_hbm, out_hbm, idx_v, rows_v,
                      gsem0, gsem1, osem0, osem1):
        gsem = (gsem0, gsem1)
        osem = (osem0, osem1)
        wid = lax.axis_index("s") * NC + lax.axis_index("c")
        row0 = wid * rows_per_w
        out0 = wid * b_per_w

        def load_chunk(ch, buf):
            pltpu.sync_copy(idx_hbm.at[pl.ds((row0 + ch * k) * G, c)],
                            idx_v.at[buf])
            pltpu.async_copy(table_hbm.at[idx_v.at[buf]],
                             rows_v.at[buf], gsem[buf])

        def drain_gathers(buf):
            pltpu.make_async_copy(table_hbm.at[idx_v.at[buf]],
                                  rows_v.at[buf], gsem[buf]).wait()

        def start_store(ch, buf):
            return pltpu.async_copy(rows_v.at[buf],
                                    out_hbm.at[pl.ds(out0 + ch * c, c)],
                                    osem[buf])

        def wait_store(ch, buf):
            pltpu.make_async_copy(rows_v.at[buf],
                                  out_hbm.at[pl.ds(out0 + ch * c, c)],
                                  osem[buf]).wait()

        def block(ch, b):
            wait_store(ch - 2, b)
            load_chunk(ch, b)
            drain_gathers(1 - b)
            start_store(ch - 1, 1 - b)

        load_chunk(0, 0)
        load_chunk(1, 1)
        drain_gathers(0)
        start_store(0, 0)

        n_even = n_chunks if n_chunks % 2 == 0 else n_chunks - 1

        def body(g):
            block(g, 0)
            block(g + 1, 1)

        pl.loop(2, n_even, step=2)(body)

        if n_chunks % 2:
            block(n_chunks - 1, 0)

        last = n_chunks - 1
        bl = last & 1
        wait_store(last - 1, 1 - bl)
        drain_gathers(bl)
        start_store(last, bl)
        wait_store(last, bl)

    return gather_kernel


def kernel(entity_tok, table):
    batch, hist = entity_tok.shape
    vocab, embed = table.shape
    b_total = batch * hist
    idx = entity_tok.reshape(b_total).astype(jnp.int32)
    out = _make_gather(vocab, embed, b_total)(idx, table)
    return out.reshape(batch, hist, embed)


# speedup vs baseline: 1.1096x; 1.0023x over previous
import functools

import jax
import jax.numpy as jnp
from jax import lax
from jax.experimental import pallas as pl
from jax.experimental.pallas import tpu as pltpu
from jax.experimental.pallas import tpu_sc as plsc

NC = 2
NS = 16
NW = NC * NS
G = 128


@functools.lru_cache(maxsize=None)
def _make_gather(vocab: int, embed: int, b_total: int):
    assert b_total % (NW * G) == 0
    b_per_w = b_total // NW
    rows_per_w = b_per_w // G
    c = 1600
    assert b_per_w % c == 0 and c % 8 == 0
    n_chunks = b_per_w // c

    mesh = plsc.VectorSubcoreMesh(core_axis_name="c", subcore_axis_name="s")

    @functools.partial(
        pl.kernel,
        mesh=mesh,
        compiler_params=pltpu.CompilerParams(use_tc_tiling_on_sc=False),
        out_type=jax.ShapeDtypeStruct((b_total, embed), jnp.float32),
        scratch_types=[
            pltpu.VMEM((2, c), jnp.int32),
            pltpu.VMEM((2, c, embed), jnp.float32),
            pltpu.SemaphoreType.DMA,
            pltpu.SemaphoreType.DMA,
            pltpu.SemaphoreType.DMA,
            pltpu.SemaphoreType.DMA,
        ],
    )
    def gather_kernel(idx_hbm, table_hbm, out_hbm, idx_v, rows_v,
                      gsem0, gsem1, osem0, osem1):
        gsem = (gsem0, gsem1)
        osem = (osem0, osem1)
        wid = lax.axis_index("s") * NC + lax.axis_index("c")
        out0 = wid * b_per_w

        def load_chunk(ch, buf):
            pltpu.sync_copy(idx_hbm.at[pl.ds(out0 + ch * c, c)],
                            idx_v.at[buf])
            pltpu.async_copy(table_hbm.at[idx_v.at[buf]],
                             rows_v.at[buf], gsem[buf])

        def drain_gathers(buf):
            pltpu.make_async_copy(table_hbm.at[idx_v.at[buf]],
                                  rows_v.at[buf], gsem[buf]).wait()

        def start_store(ch, buf):
            return pltpu.async_copy(rows_v.at[buf],
                                    out_hbm.at[pl.ds(out0 + ch * c, c)],
                                    osem[buf])

        def wait_store(ch, buf):
            pltpu.make_async_copy(rows_v.at[buf],
                                  out_hbm.at[pl.ds(out0 + ch * c, c)],
                                  osem[buf]).wait()

        def block(ch, b):
            wait_store(ch - 2, b)
            load_chunk(ch, b)
            drain_gathers(1 - b)
            start_store(ch - 1, 1 - b)

        load_chunk(0, 0)
        load_chunk(1, 1)
        drain_gathers(0)
        start_store(0, 0)

        n_even = n_chunks if n_chunks % 2 == 0 else n_chunks - 1

        def body(g):
            block(g, 0)
            block(g + 1, 1)

        pl.loop(2, n_even, step=2)(body)

        if n_chunks % 2:
            block(n_chunks - 1, 0)

        last = n_chunks - 1
        bl = last & 1
        wait_store(last - 1, 1 - bl)
        drain_gathers(bl)
        start_store(last, bl)
        wait_store(last, bl)

    return gather_kernel


def kernel(entity_tok, table):
    batch, hist = entity_tok.shape
    vocab, embed = table.shape
    b_total = batch * hist
    idx = entity_tok.reshape(b_total).astype(jnp.int32)
    out = _make_gather(vocab, embed, b_total)(idx, table)
    return out.reshape(batch, hist, embed)
